# baseline (speedup 1.0000x reference)
import jax
import jax.numpy as jnp
from jax.experimental import pallas as pl
from jax.experimental.pallas import tpu as pltpu

_NP = 98
_HID = 128
_NL = 4
_INCH = 256
_HW = 32 * 32
_NB = 4

_F = jnp.float32


def _dot(a, b):
    return jnp.dot(a, b, preferred_element_type=_F)


def _fused_kernel(x_ref, cw_ref, cb_ref, win_ref, bin_ref,
                  wq_ref, wk_ref, wg_ref, bg_ref, wout_ref, bout_ref,
                  out_ref):
    scale = 1.0 / jnp.sqrt(_F(_HID))
    accs = [jnp.zeros((_NP, _HID), _F) for _ in range(_NB)]
    for j in range(4):
        hs = [_dot(cw_ref[j], x_ref[s]) for s in range(_NB)]
        hs = [jnp.maximum(h + cb_ref[j], 0.0) for h in hs]
        accs = [acc + _dot(h, win_ref[j]) for acc, h in zip(accs, hs)]
    xs = [jnp.maximum(acc + bin_ref[...], 0.0) for acc in accs]
    for l in range(_NL):
        qs = [_dot(x, wq_ref[l]) for x in xs]
        ks = [_dot(x, wk_ref[l]) for x in xs]
        gs = [_dot(x, wg_ref[l]) for x in xs]
        ls_ = [jax.lax.dot_general(q, k, (((1,), (1,)), ((), ())),
                                   preferred_element_type=_F) * scale
               for q, k in zip(qs, ks)]
        as_ = [jax.nn.softmax(lg, axis=-1) for lg in ls_]
        msgs = [_dot(a, g) + bg_ref[l] for a, g in zip(as_, gs)]
        xs = [jnp.maximum(m + x, 0.0) for m, x in zip(msgs, xs)]
    for s in range(_NB):
        out_ref[s] = _dot(xs[s], wout_ref[...]) + bout_ref[...]


def kernel(x_dict, conv_w, conv_b, W_in, b_in, Wq, Wk, Wg, bg, W_out, b_out):
    b = x_dict.shape[0]
    xr = x_dict.reshape(b, _INCH, _HW)
    cw_r = conv_w.reshape(_NP, 4, _INCH).transpose(1, 0, 2)
    cb_r = conv_b.reshape(_NP, 4).T.reshape(4, _NP, 1)
    win_r = W_in.reshape(4, _HW, _HID)
    bin_r = b_in.reshape(1, _HID)
    bg_r = bg.reshape(_NL, 1, _HID)
    wout_p = jnp.zeros((_HID, _HID), _F).at[:, :2].set(W_out)
    bout_p = jnp.zeros((1, _HID), _F).at[0, :2].set(b_out)

    out = pl.pallas_call(
        _fused_kernel,
        grid=(b // _NB,),
        compiler_params=pltpu.CompilerParams(
            dimension_semantics=("arbitrary",)),
        in_specs=[
            pl.BlockSpec((_NB, _INCH, _HW), lambda i: (i, 0, 0)),
            pl.BlockSpec((4, _NP, _INCH), lambda i: (0, 0, 0)),
            pl.BlockSpec((4, _NP, 1), lambda i: (0, 0, 0)),
            pl.BlockSpec((4, _HW, _HID), lambda i: (0, 0, 0)),
            pl.BlockSpec((1, _HID), lambda i: (0, 0)),
            pl.BlockSpec((_NL, _HID, _HID), lambda i: (0, 0, 0)),
            pl.BlockSpec((_NL, _HID, _HID), lambda i: (0, 0, 0)),
            pl.BlockSpec((_NL, _HID, _HID), lambda i: (0, 0, 0)),
            pl.BlockSpec((_NL, 1, _HID), lambda i: (0, 0, 0)),
            pl.BlockSpec((_HID, _HID), lambda i: (0, 0)),
            pl.BlockSpec((1, _HID), lambda i: (0, 0)),
        ],
        out_specs=pl.BlockSpec((_NB, _NP, _HID), lambda i: (i, 0, 0)),
        out_shape=jax.ShapeDtypeStruct((b, _NP, _HID), jnp.float32),
    )(xr, cw_r, cb_r, win_r, bin_r, Wq, Wk, Wg, bg_r, wout_p, bout_p)
    return out[:, :, :2].reshape(b, -1)


# speedup vs baseline: 1.1147x; 1.1147x over previous
import jax
import jax.numpy as jnp
from jax.experimental import pallas as pl
from jax.experimental.pallas import tpu as pltpu

_NP = 98
_HID = 128
_NL = 4
_INCH = 256
_HW = 32 * 32
_NB = 16

_F = jnp.float32


def _dot(a, b):
    return jnp.dot(a, b, preferred_element_type=_F)


def _fused_kernel(x_ref, cw_ref, cb_ref, win_ref, bin_ref,
                  wq_ref, wk_ref, wg_ref, bg_ref, wout_ref, bout_ref,
                  out_ref):
    scale = 1.0 / jnp.sqrt(_F(_HID))
    accs = [jnp.zeros((_NP, _HID), _F) for _ in range(_NB)]
    for j in range(4):
        hs = [_dot(cw_ref[j], x_ref[s]) for s in range(_NB)]
        hs = [jnp.maximum(h + cb_ref[j], 0.0).astype(jnp.bfloat16)
              for h in hs]
        accs = [acc + _dot(h, win_ref[j]) for acc, h in zip(accs, hs)]
    xs = [jnp.maximum(acc + bin_ref[...], 0.0) for acc in accs]
    for l in range(_NL):
        qs = [_dot(x, wq_ref[l]) for x in xs]
        ks = [_dot(x, wk_ref[l]) for x in xs]
        gs = [_dot(x, wg_ref[l]) for x in xs]
        ls_ = [jax.lax.dot_general(q, k, (((1,), (1,)), ((), ())),
                                   preferred_element_type=_F) * scale
               for q, k in zip(qs, ks)]
        as_ = [jax.nn.softmax(lg, axis=-1) for lg in ls_]
        msgs = [_dot(a, g) + bg_ref[l] for a, g in zip(as_, gs)]
        xs = [jnp.maximum(m + x, 0.0) for m, x in zip(msgs, xs)]
    for s in range(_NB):
        out_ref[s] = _dot(xs[s], wout_ref[...]) + bout_ref[...]


def kernel(x_dict, conv_w, conv_b, W_in, b_in, Wq, Wk, Wg, bg, W_out, b_out):
    b = x_dict.shape[0]
    xr = x_dict.reshape(b, _INCH, _HW)
    cw_r = conv_w.reshape(_NP, 4, _INCH).transpose(1, 0, 2)
    cb_r = conv_b.reshape(_NP, 4).T.reshape(4, _NP, 1)
    win_r = W_in.reshape(4, _HW, _HID).astype(jnp.bfloat16)
    bin_r = b_in.reshape(1, _HID)
    bg_r = bg.reshape(_NL, 1, _HID)
    wout_p = jnp.zeros((_HID, _HID), _F).at[:, :2].set(W_out)
    bout_p = jnp.zeros((1, _HID), _F).at[0, :2].set(b_out)

    out = pl.pallas_call(
        _fused_kernel,
        grid=(b // _NB,),
        compiler_params=pltpu.CompilerParams(
            dimension_semantics=("parallel",)),
        in_specs=[
            pl.BlockSpec((_NB, _INCH, _HW), lambda i: (i, 0, 0)),
            pl.BlockSpec((4, _NP, _INCH), lambda i: (0, 0, 0)),
            pl.BlockSpec((4, _NP, 1), lambda i: (0, 0, 0)),
            pl.BlockSpec((4, _HW, _HID), lambda i: (0, 0, 0)),
            pl.BlockSpec((1, _HID), lambda i: (0, 0)),
            pl.BlockSpec((_NL, _HID, _HID), lambda i: (0, 0, 0)),
            pl.BlockSpec((_NL, _HID, _HID), lambda i: (0, 0, 0)),
            pl.BlockSpec((_NL, _HID, _HID), lambda i: (0, 0, 0)),
            pl.BlockSpec((_NL, 1, _HID), lambda i: (0, 0, 0)),
            pl.BlockSpec((_HID, _HID), lambda i: (0, 0)),
            pl.BlockSpec((1, _HID), lambda i: (0, 0)),
        ],
        out_specs=pl.BlockSpec((_NB, _NP, _HID), lambda i: (i, 0, 0)),
        out_shape=jax.ShapeDtypeStruct((b, _NP, _HID), jnp.float32),
    )(xr, cw_r, cb_r, win_r, bin_r, Wq, Wk, Wg, bg_r, wout_p, bout_p)
    return out[:, :, :2].reshape(b, -1)


# speedup vs baseline: 1.1724x; 1.0518x over previous
import jax
import jax.numpy as jnp
from jax.experimental import pallas as pl
from jax.experimental.pallas import tpu as pltpu

_NP = 98
_HID = 128
_NL = 4
_INCH = 256
_HW = 32 * 32
_NB = 16

_F = jnp.float32


def _dot(a, b):
    return jnp.dot(a, b, preferred_element_type=_F)


def _fused_kernel(x_ref, cw_ref, cb_ref, win_ref, bin_ref,
                  wqkg_ref, bg_ref, wout_ref, bout_ref,
                  out_ref):
    accs = [jnp.zeros((_NP, _HID), _F) for _ in range(_NB)]
    for j in range(4):
        hs = [jnp.dot(cw_ref[j], x_ref[s], preferred_element_type=_F)
              for s in range(_NB)]
        hs = [jnp.maximum(h + cb_ref[j], 0.0) for h in hs]
        accs = [acc + jnp.dot(h, win_ref[j], preferred_element_type=_F)
                for acc, h in zip(accs, hs)]
    xs = [jnp.maximum(acc + bin_ref[...], 0.0) for acc in accs]
    for l in range(_NL):
        qkgs = [jnp.dot(x, wqkg_ref[l], preferred_element_type=_F)
                for x in xs]
        ls_ = [jax.lax.dot_general(qkg[:, :_HID], qkg[:, _HID:2 * _HID],
                                   (((1,), (1,)), ((), ())),
                                   preferred_element_type=_F)
               for qkg in qkgs]
        es = [jnp.exp(lg) for lg in ls_]
        as_ = [e / jnp.sum(e, axis=-1, keepdims=True) for e in es]
        msgs = [jnp.dot(a, qkg[:, 2 * _HID:], preferred_element_type=_F)
                + bg_ref[l] for a, qkg in zip(as_, qkgs)]
        xs = [jnp.maximum(m + x, 0.0) for m, x in zip(msgs, xs)]
    for s in range(_NB):
        out_ref[s] = jnp.dot(xs[s], wout_ref[...], preferred_element_type=_F) + bout_ref[...]


def kernel(x_dict, conv_w, conv_b, W_in, b_in, Wq, Wk, Wg, bg, W_out, b_out):
    b = x_dict.shape[0]
    xr = x_dict.reshape(b, _INCH, _HW)
    cw_r = conv_w.reshape(_NP, 4, _INCH).transpose(1, 0, 2)
    cb_r = conv_b.reshape(_NP, 4).T.reshape(4, _NP, 1)
    win_r = W_in.reshape(4, _HW, _HID)
    bin_r = b_in.reshape(1, _HID)
    scale = 1.0 / jnp.sqrt(jnp.array(_HID, _F))
    wqkg = jnp.concatenate([Wq * scale, Wk, Wg], axis=2)
    bg_r = bg.reshape(_NL, 1, _HID)
    wout_p = jnp.zeros((_HID, _HID), _F).at[:, :2].set(W_out)
    bout_p = jnp.zeros((1, _HID), _F).at[0, :2].set(b_out)

    out = pl.pallas_call(
        _fused_kernel,
        grid=(b // _NB,),
        compiler_params=pltpu.CompilerParams(
            dimension_semantics=("parallel",)),
        in_specs=[
            pl.BlockSpec((_NB, _INCH, _HW), lambda i: (i, 0, 0)),
            pl.BlockSpec((4, _NP, _INCH), lambda i: (0, 0, 0)),
            pl.BlockSpec((4, _NP, 1), lambda i: (0, 0, 0)),
            pl.BlockSpec((4, _HW, _HID), lambda i: (0, 0, 0)),
            pl.BlockSpec((1, _HID), lambda i: (0, 0)),
            pl.BlockSpec((_NL, _HID, 3 * _HID), lambda i: (0, 0, 0)),
            pl.BlockSpec((_NL, 1, _HID), lambda i: (0, 0, 0)),
            pl.BlockSpec((_HID, _HID), lambda i: (0, 0)),
            pl.BlockSpec((1, _HID), lambda i: (0, 0)),
        ],
        out_specs=pl.BlockSpec((_NB, _NP, _HID), lambda i: (i, 0, 0)),
        out_shape=jax.ShapeDtypeStruct((b, _NP, _HID), jnp.float32),
    )(xr, cw_r, cb_r, win_r, bin_r, wqkg, bg_r, wout_p, bout_p)
    return out[:, :, :2].reshape(b, -1)


# speedup vs baseline: 1.2073x; 1.0298x over previous
import jax
import jax.numpy as jnp
from jax.experimental import pallas as pl
from jax.experimental.pallas import tpu as pltpu

_NP = 98
_NG = 104
_HID = 128
_NL = 4
_INCH = 256
_HW = 32 * 32
_NB = 16
_NW = 8

_F = jnp.float32


def _fused_kernel(x_ref, cw_ref, cb_ref, win_ref, bin_ref,
                  wqkg_ref, bg_ref, wout_ref, bout_ref,
                  out_ref):
    rowmask = jax.lax.broadcasted_iota(jnp.int32, (_NG, _HID), 0) < _NP
    colmask = jax.lax.broadcasted_iota(jnp.int32, (_NG, _NG), 1) < _NP
    xs = []
    for w in range(0, _NB, _NW):
        hs = [jnp.dot(cw_ref[...], x_ref[s], preferred_element_type=_F)
              for s in range(w, w + _NW)]
        hs = [jnp.maximum(h + cb_ref[...], 0.0) for h in hs]
        for h in hs:
            acc = jnp.dot(h[0:_NG], win_ref[0], preferred_element_type=_F)
            for j in range(1, 4):
                acc = acc + jnp.dot(h[j * _NG:(j + 1) * _NG], win_ref[j],
                                    preferred_element_type=_F)
            xs.append(acc)
    xs = [jnp.where(rowmask, jnp.maximum(x + bin_ref[...], 0.0), 0.0)
          for x in xs]
    for l in range(_NL):
        qkgs = [jnp.dot(x, wqkg_ref[l], preferred_element_type=_F)
                for x in xs]
        ls_ = [jax.lax.dot_general(qkg[:, :_HID], qkg[:, _HID:2 * _HID],
                                   (((1,), (1,)), ((), ())),
                                   preferred_element_type=_F)
               for qkg in qkgs]
        es = [jnp.where(colmask, jnp.exp(lg), 0.0) for lg in ls_]
        as_ = [e / jnp.sum(e, axis=-1, keepdims=True) for e in es]
        msgs = [jnp.dot(a, qkg[:, 2 * _HID:], preferred_element_type=_F)
                + bg_ref[l] for a, qkg in zip(as_, qkgs)]
        xs = [jnp.maximum(m + x, 0.0) for m, x in zip(msgs, xs)]
    for s in range(_NB):
        out_ref[s] = (jnp.dot(xs[s], wout_ref[...], preferred_element_type=_F)
                      + bout_ref[...])


def kernel(x_dict, conv_w, conv_b, W_in, b_in, Wq, Wk, Wg, bg, W_out, b_out):
    b = x_dict.shape[0]
    xr = x_dict.reshape(b, _INCH, _HW)
    cw_g = conv_w.reshape(_NP, 4, _INCH).transpose(1, 0, 2)
    cw_p = jnp.zeros((4, _NG, _INCH), _F).at[:, :_NP].set(cw_g)
    cw_p = cw_p.reshape(4 * _NG, _INCH)
    cb_g = conv_b.reshape(_NP, 4).T
    cb_p = jnp.zeros((4, _NG), _F).at[:, :_NP].set(cb_g)
    cb_p = cb_p.reshape(4 * _NG, 1)
    win_r = W_in.reshape(4, _HW, _HID)
    bin_r = b_in.reshape(1, _HID)
    scale = 1.0 / jnp.sqrt(jnp.array(_HID, _F))
    wqkg = jnp.concatenate([Wq * scale, Wk, Wg], axis=2)
    bg_r = bg.reshape(_NL, 1, _HID)
    wout_p = jnp.zeros((_HID, _HID), _F).at[:, :2].set(W_out)
    bout_p = jnp.zeros((1, _HID), _F).at[0, :2].set(b_out)

    out = pl.pallas_call(
        _fused_kernel,
        grid=(b // _NB,),
        compiler_params=pltpu.CompilerParams(
            dimension_semantics=("parallel",)),
        in_specs=[
            pl.BlockSpec((_NB, _INCH, _HW), lambda i: (i, 0, 0)),
            pl.BlockSpec((4 * _NG, _INCH), lambda i: (0, 0)),
            pl.BlockSpec((4 * _NG, 1), lambda i: (0, 0)),
            pl.BlockSpec((4, _HW, _HID), lambda i: (0, 0, 0)),
            pl.BlockSpec((1, _HID), lambda i: (0, 0)),
            pl.BlockSpec((_NL, _HID, 3 * _HID), lambda i: (0, 0, 0)),
            pl.BlockSpec((_NL, 1, _HID), lambda i: (0, 0, 0)),
            pl.BlockSpec((_HID, _HID), lambda i: (0, 0)),
            pl.BlockSpec((1, _HID), lambda i: (0, 0)),
        ],
        out_specs=pl.BlockSpec((_NB, _NG, _HID), lambda i: (i, 0, 0)),
        out_shape=jax.ShapeDtypeStruct((b, _NG, _HID), jnp.float32),
    )(xr, cw_p, cb_p, win_r, bin_r, wqkg, bg_r, wout_p, bout_p)
    return out[:, :_NP, :2].reshape(b, -1)
